# baseline (speedup 1.0000x reference)
import functools

import jax
import jax.numpy as jnp
from jax.experimental import pallas as pl

K_NEIGH = 16
NUM_CLS = 4
N_CLUSTERS = 2
EPS = 1e-8

Q = 4096
D = 128
M = 100000
MP = 100352
QT = 256
B = 2048


def _dist_kernel(x_ref, xm_ref, q2_ref, m2_ref, o_ref):
    dot = jax.lax.dot_general(
        x_ref[...], xm_ref[...],
        dimension_numbers=(((1,), (1,)), ((), ())),
        preferred_element_type=jnp.float32,
    )
    o_ref[...] = (q2_ref[...] + m2_ref[...]) - 2.0 * dot


def _cent_kernel(x_ref, c_ref, q2_ref, c2_ref, o_ref):
    dot = jax.lax.dot_general(
        x_ref[...], c_ref[...],
        dimension_numbers=(((1,), (1,)), ((), ())),
        preferred_element_type=jnp.float32,
    )
    dc = (q2_ref[...] + c2_ref[...]) - 2.0 * dot
    dc = dc.reshape(x_ref.shape[0], NUM_CLS, N_CLUSTERS)
    o_ref[...] = jnp.min(dc, axis=2)


def kernel(X, X_mem, y_mem, centroids):
    q2 = jnp.sum(X * X, axis=1, keepdims=True)
    m2 = jnp.sum(X_mem * X_mem, axis=1)
    m2p = jnp.full((MP,), 1e30, dtype=jnp.float32).at[:M].set(m2)
    xmp = jnp.zeros((MP, D), dtype=jnp.float32).at[:M].set(X_mem)

    d2 = pl.pallas_call(
        _dist_kernel,
        grid=(Q // QT, MP // B),
        in_specs=[
            pl.BlockSpec((QT, D), lambda i, j: (i, 0)),
            pl.BlockSpec((B, D), lambda i, j: (j, 0)),
            pl.BlockSpec((QT, 1), lambda i, j: (i, 0)),
            pl.BlockSpec((1, B), lambda i, j: (0, j)),
        ],
        out_specs=pl.BlockSpec((QT, B), lambda i, j: (i, j)),
        out_shape=jax.ShapeDtypeStruct((Q, MP), jnp.float32),
    )(X, xmp, q2, m2p.reshape(1, MP))

    neg_d, idx = jax.lax.top_k(-d2, K_NEIGH)
    nd = jnp.maximum(-neg_d, 0.0)
    labels = jnp.take(y_mem, idx, axis=0)
    w = 1.0 / (nd + EPS)
    onehot = jax.nn.one_hot(labels, NUM_CLS, dtype=X.dtype)
    knn_scores = jnp.sum(w[..., None] * onehot, axis=1)
    knn_scores = knn_scores / (jnp.sum(w, axis=1, keepdims=True) + EPS)

    c2 = jnp.sum(centroids * centroids, axis=1)
    cent_dist = pl.pallas_call(
        _cent_kernel,
        grid=(Q // QT,),
        in_specs=[
            pl.BlockSpec((QT, D), lambda i: (i, 0)),
            pl.BlockSpec((NUM_CLS * N_CLUSTERS, D), lambda i: (0, 0)),
            pl.BlockSpec((QT, 1), lambda i: (i, 0)),
            pl.BlockSpec((1, NUM_CLS * N_CLUSTERS), lambda i: (0, 0)),
        ],
        out_specs=pl.BlockSpec((QT, NUM_CLS), lambda i: (i, 0)),
        out_shape=jax.ShapeDtypeStruct((Q, NUM_CLS), jnp.float32),
    )(X, centroids, q2, c2.reshape(1, NUM_CLS * N_CLUSTERS))

    knn_pred = jnp.argmax(knn_scores, axis=1).astype(jnp.int32)
    return knn_pred, knn_scores, cent_dist


# speedup vs baseline: 6.8708x; 6.8708x over previous
import jax
import jax.numpy as jnp
from jax import lax
from jax.experimental import pallas as pl
from jax.experimental.pallas import tpu as pltpu
from jax.experimental.pallas import tpu_sc as plsc

K_NEIGH = 16
NUM_CLS = 4
N_CLUSTERS = 2
R_EXP = 1.0
EPS = 1e-8

Q = 4096
D = 128
M = 100000
B = 2048
NBLK = 49
MP = NBLK * B
QT = 256
GSZ = 128
NG = MP // GSZ
NCAND = K_NEIGH * GSZ
NROWS = K_NEIGH * Q
GW = 128

BIGF = 3e38
BIGI = 2**31 - 1


def _dist_colmin_kernel(x_ref, xm_ref, q2_ref, m2_ref, d_ref, cm_ref):
    dot = lax.dot_general(
        x_ref[...], xm_ref[...],
        dimension_numbers=(((1,), (1,)), ((), ())),
        preferred_element_type=jnp.float32,
    )
    d2 = (q2_ref[...] + m2_ref[...]) - 2.0 * dot
    d_ref[...] = d2
    cm = jnp.concatenate(
        [jnp.min(d2[:, c * 128:(c + 1) * 128], axis=1, keepdims=True)
         for c in range(B // 128)], axis=1)
    cm_ref[...] = cm.T


def _group_topk_kernel(cm_ref, g_ref):
    val = cm_ref[...]
    ridx = lax.broadcasted_iota(jnp.int32, (NG, QT), 0)
    for p in range(K_NEIGH):
        m = jnp.min(val, axis=0, keepdims=True)
        elig = val <= m
        ai = jnp.min(jnp.where(elig, ridx, BIGI), axis=0, keepdims=True)
        g_ref[p, :] = ai[0, :]
        val = jnp.where(ridx == ai, BIGF, val)


def _cand_topk_kernel(cv_ref, g_ref, nd_ref, js_ref):
    val = cv_ref[...]
    lane = lax.broadcasted_iota(jnp.int32, (QT, 128), 1)
    jv = jnp.concatenate(
        [g_ref[:, p:p + 1] * 128 + lane for p in range(K_NEIGH)],
        axis=1)
    for p in range(K_NEIGH):
        m = jnp.min(val, axis=1, keepdims=True)
        elig = val <= m
        aj = jnp.min(jnp.where(elig, jv, BIGI), axis=1, keepdims=True)
        nd_ref[:, p] = m[:, 0]
        js_ref[:, p] = aj[:, 0]
        val = jnp.where(jv == aj, BIGF, val)


def _cent_kernel(x_ref, c_ref, q2_ref, c2_ref, o_ref):
    dot = lax.dot_general(
        x_ref[...], c_ref[...],
        dimension_numbers=(((1,), (1,)), ((), ())),
        preferred_element_type=jnp.float32,
    )
    dc = (q2_ref[...] + c2_ref[...]) - 2.0 * dot
    dc = dc.reshape(x_ref.shape[0], NUM_CLS, N_CLUSTERS)
    o_ref[...] = jnp.min(dc, axis=2)


def _gather_rows(d2_rows, row_ids):
    mesh = plsc.VectorSubcoreMesh(core_axis_name="c", subcore_axis_name="s")

    @pl.kernel(out_type=jax.ShapeDtypeStruct((NROWS, GSZ), jnp.float32),
               mesh=mesh)
    def sc_gather(x_hbm, i_hbm, o_hbm):
        def body(i_vmem, o_vmem):
            pltpu.sync_copy(x_hbm.at[i_vmem.at[0]], o_vmem)

        pltpu.emit_pipeline(
            body,
            grid=(NROWS // GW,),
            in_specs=[pl.BlockSpec((1, GW), lambda i: (0, i))],
            out_specs=[pl.BlockSpec((GW, GSZ), lambda i: (i, 0))],
            core_axis_name=("c", "s"),
            dimension_semantics=(pltpu.PARALLEL,),
        )(i_hbm, o_hbm)

    return sc_gather(d2_rows, row_ids)


def kernel(X, X_mem, y_mem, centroids):
    q2 = jnp.sum(X * X, axis=1, keepdims=True)
    m2 = jnp.sum(X_mem * X_mem, axis=1)
    m2p = jnp.full((MP,), 1e30, dtype=jnp.float32).at[:M].set(m2)
    xmp = jnp.zeros((MP, D), dtype=jnp.float32).at[:M].set(X_mem)

    d2, colmin_t = pl.pallas_call(
        _dist_colmin_kernel,
        grid=(Q // QT, NBLK),
        in_specs=[
            pl.BlockSpec((QT, D), lambda i, j: (i, 0)),
            pl.BlockSpec((B, D), lambda i, j: (j, 0)),
            pl.BlockSpec((QT, 1), lambda i, j: (i, 0)),
            pl.BlockSpec((1, B), lambda i, j: (0, j)),
        ],
        out_specs=[
            pl.BlockSpec((QT, B), lambda i, j: (i, j)),
            pl.BlockSpec((B // 128, QT), lambda i, j: (j, i)),
        ],
        out_shape=[
            jax.ShapeDtypeStruct((Q, MP), jnp.float32),
            jax.ShapeDtypeStruct((NG, Q), jnp.float32),
        ],
    )(X, xmp, q2, m2p.reshape(1, MP))

    gids_t = pl.pallas_call(
        _group_topk_kernel,
        grid=(Q // QT,),
        in_specs=[pl.BlockSpec((NG, QT), lambda i: (0, i))],
        out_specs=pl.BlockSpec((K_NEIGH, QT), lambda i: (0, i)),
        out_shape=jax.ShapeDtypeStruct((K_NEIGH, Q), jnp.int32),
    )(colmin_t)
    gids = gids_t.T

    row_ids = jnp.arange(Q, dtype=jnp.int32)[:, None] * NG + gids
    cand_v = _gather_rows(
        d2.reshape(Q * MP // GSZ, GSZ), row_ids.reshape(1, NROWS)
    ).reshape(Q, NCAND)

    ndv, jsel = pl.pallas_call(
        _cand_topk_kernel,
        grid=(Q // QT,),
        in_specs=[
            pl.BlockSpec((QT, NCAND), lambda i: (i, 0)),
            pl.BlockSpec((QT, K_NEIGH), lambda i: (i, 0)),
        ],
        out_specs=[
            pl.BlockSpec((QT, K_NEIGH), lambda i: (i, 0)),
            pl.BlockSpec((QT, K_NEIGH), lambda i: (i, 0)),
        ],
        out_shape=[
            jax.ShapeDtypeStruct((Q, K_NEIGH), jnp.float32),
            jax.ShapeDtypeStruct((Q, K_NEIGH), jnp.int32),
        ],
    )(cand_v, gids)

    nd = jnp.maximum(ndv, 0.0)
    labels = jnp.take(y_mem, jsel, axis=0)
    w = 1.0 / (nd + EPS) ** R_EXP
    onehot = jax.nn.one_hot(labels, NUM_CLS, dtype=X.dtype)
    knn_scores = jnp.sum(w[..., None] * onehot, axis=1)
    knn_scores = knn_scores / (jnp.sum(w, axis=1, keepdims=True) + EPS)

    c2 = jnp.sum(centroids * centroids, axis=1)
    cent_dist = pl.pallas_call(
        _cent_kernel,
        grid=(Q // QT,),
        in_specs=[
            pl.BlockSpec((QT, D), lambda i: (i, 0)),
            pl.BlockSpec((NUM_CLS * N_CLUSTERS, D), lambda i: (0, 0)),
            pl.BlockSpec((QT, 1), lambda i: (i, 0)),
            pl.BlockSpec((1, NUM_CLS * N_CLUSTERS), lambda i: (0, 0)),
        ],
        out_specs=pl.BlockSpec((QT, NUM_CLS), lambda i: (i, 0)),
        out_shape=jax.ShapeDtypeStruct((Q, NUM_CLS), jnp.float32),
    )(X, centroids, q2, c2.reshape(1, NUM_CLS * N_CLUSTERS))

    knn_pred = jnp.argmax(knn_scores, axis=1).astype(jnp.int32)
    return knn_pred, knn_scores, cent_dist


# speedup vs baseline: 9.5139x; 1.3847x over previous
import jax
import jax.numpy as jnp
from jax import lax
from jax.experimental import pallas as pl
from jax.experimental.pallas import tpu as pltpu
from jax.experimental.pallas import tpu_sc as plsc

K_NEIGH = 16
NUM_CLS = 4
N_CLUSTERS = 2
R_EXP = 1.0
EPS = 1e-8

Q = 4096
D = 128
M = 100000
B = 2048
NBLK = 49
MP = NBLK * B
QT = 256
GSZ = 128
NG = MP // GSZ
NCAND = K_NEIGH * GSZ
NROWS = K_NEIGH * Q
GW = 128

BIGF = 3e38
BIGI = 2**31 - 1


def _dist_colmin_kernel(x_ref, xm_ref, q2_ref, m2_ref, d_ref, cm_ref):
    dot = lax.dot_general(
        x_ref[...], xm_ref[...],
        dimension_numbers=(((1,), (1,)), ((), ())),
        preferred_element_type=jnp.float32,
    )
    d2 = (q2_ref[...] + m2_ref[...]) - 2.0 * dot
    d_ref[...] = d2.reshape(QT // 8, 8, B // 128, 128).transpose(0, 2, 1, 3)
    cm = jnp.concatenate(
        [jnp.min(d2[:, c * 128:(c + 1) * 128], axis=1, keepdims=True)
         for c in range(B // 128)], axis=1)
    cm_ref[...] = cm.T


def _group_topk_kernel(cm_ref, g_ref):
    val = cm_ref[...]
    ridx = lax.broadcasted_iota(jnp.int32, (NG, QT), 0)
    for p in range(K_NEIGH):
        m = jnp.min(val, axis=0, keepdims=True)
        elig = val <= m
        ai = jnp.min(jnp.where(elig, ridx, BIGI), axis=0, keepdims=True)
        g_ref[p, :] = ai[0, :]
        val = jnp.where(ridx == ai, BIGF, val)


def _cand_topk_kernel(cv_ref, g_ref, nd_ref, js_ref):
    val = cv_ref[...]
    lane = lax.broadcasted_iota(jnp.int32, (QT, 128), 1)
    jv = jnp.concatenate(
        [g_ref[:, p:p + 1] * 128 + lane for p in range(K_NEIGH)],
        axis=1)
    for p in range(K_NEIGH):
        m = jnp.min(val, axis=1, keepdims=True)
        elig = val <= m
        aj = jnp.min(jnp.where(elig, jv, BIGI), axis=1, keepdims=True)
        nd_ref[:, p] = m[:, 0]
        js_ref[:, p] = aj[:, 0]
        val = jnp.where(jv == aj, BIGF, val)


def _cent_kernel(x_ref, c_ref, q2_ref, c2_ref, o_ref):
    dot = lax.dot_general(
        x_ref[...], c_ref[...],
        dimension_numbers=(((1,), (1,)), ((), ())),
        preferred_element_type=jnp.float32,
    )
    dc = (q2_ref[...] + c2_ref[...]) - 2.0 * dot
    dc = dc.reshape(x_ref.shape[0], NUM_CLS, N_CLUSTERS)
    o_ref[...] = jnp.min(dc, axis=2)


def _gather_rows(d2_rows, row_ids):
    mesh = plsc.VectorSubcoreMesh(core_axis_name="c", subcore_axis_name="s")

    @pl.kernel(out_type=jax.ShapeDtypeStruct((NROWS, GSZ), jnp.float32),
               mesh=mesh)
    def sc_gather(x_hbm, i_hbm, o_hbm):
        def body(i_vmem, o_vmem):
            pltpu.sync_copy(x_hbm.at[i_vmem.at[0]], o_vmem)

        pltpu.emit_pipeline(
            body,
            grid=(NROWS // GW,),
            in_specs=[pl.BlockSpec((1, GW), lambda i: (0, i))],
            out_specs=[pl.BlockSpec((GW, GSZ), lambda i: (i, 0))],
            core_axis_name=("c", "s"),
            dimension_semantics=(pltpu.PARALLEL,),
        )(i_hbm, o_hbm)

    return sc_gather(d2_rows, row_ids)


def kernel(X, X_mem, y_mem, centroids):
    q2 = jnp.sum(X * X, axis=1, keepdims=True)
    m2 = jnp.sum(X_mem * X_mem, axis=1)
    m2p = jnp.full((MP,), 1e30, dtype=jnp.float32).at[:M].set(m2)
    xmp = jnp.zeros((MP, D), dtype=jnp.float32).at[:M].set(X_mem)

    d2, colmin_t = pl.pallas_call(
        _dist_colmin_kernel,
        grid=(Q // QT, NBLK),
        in_specs=[
            pl.BlockSpec((QT, D), lambda i, j: (i, 0)),
            pl.BlockSpec((B, D), lambda i, j: (j, 0)),
            pl.BlockSpec((QT, 1), lambda i, j: (i, 0)),
            pl.BlockSpec((1, B), lambda i, j: (0, j)),
        ],
        out_specs=[
            pl.BlockSpec((QT // 8, B // 128, 8, 128), lambda i, j: (i, j, 0, 0)),
            pl.BlockSpec((B // 128, QT), lambda i, j: (j, i)),
        ],
        out_shape=[
            jax.ShapeDtypeStruct((Q // 8, NG, 8, 128), jnp.float32),
            jax.ShapeDtypeStruct((NG, Q), jnp.float32),
        ],
    )(X, xmp, q2, m2p.reshape(1, MP))

    gids_t = pl.pallas_call(
        _group_topk_kernel,
        grid=(Q // QT,),
        in_specs=[pl.BlockSpec((NG, QT), lambda i: (0, i))],
        out_specs=pl.BlockSpec((K_NEIGH, QT), lambda i: (0, i)),
        out_shape=jax.ShapeDtypeStruct((K_NEIGH, Q), jnp.int32),
    )(colmin_t)
    gids = gids_t.T

    qv = jnp.arange(Q, dtype=jnp.int32)[:, None]
    row_ids = (qv // 8) * (NG * 8) + gids * 8 + (qv % 8)
    cand_v = _gather_rows(
        d2.reshape(Q * MP // GSZ, GSZ), row_ids.reshape(1, NROWS)
    ).reshape(Q, NCAND)

    ndv, jsel = pl.pallas_call(
        _cand_topk_kernel,
        grid=(Q // QT,),
        in_specs=[
            pl.BlockSpec((QT, NCAND), lambda i: (i, 0)),
            pl.BlockSpec((QT, K_NEIGH), lambda i: (i, 0)),
        ],
        out_specs=[
            pl.BlockSpec((QT, K_NEIGH), lambda i: (i, 0)),
            pl.BlockSpec((QT, K_NEIGH), lambda i: (i, 0)),
        ],
        out_shape=[
            jax.ShapeDtypeStruct((Q, K_NEIGH), jnp.float32),
            jax.ShapeDtypeStruct((Q, K_NEIGH), jnp.int32),
        ],
    )(cand_v, gids)

    nd = jnp.maximum(ndv, 0.0)
    labels = jnp.take(y_mem, jsel, axis=0)
    w = 1.0 / (nd + EPS) ** R_EXP
    onehot = jax.nn.one_hot(labels, NUM_CLS, dtype=X.dtype)
    knn_scores = jnp.sum(w[..., None] * onehot, axis=1)
    knn_scores = knn_scores / (jnp.sum(w, axis=1, keepdims=True) + EPS)

    c2 = jnp.sum(centroids * centroids, axis=1)
    cent_dist = pl.pallas_call(
        _cent_kernel,
        grid=(Q // QT,),
        in_specs=[
            pl.BlockSpec((QT, D), lambda i: (i, 0)),
            pl.BlockSpec((NUM_CLS * N_CLUSTERS, D), lambda i: (0, 0)),
            pl.BlockSpec((QT, 1), lambda i: (i, 0)),
            pl.BlockSpec((1, NUM_CLS * N_CLUSTERS), lambda i: (0, 0)),
        ],
        out_specs=pl.BlockSpec((QT, NUM_CLS), lambda i: (i, 0)),
        out_shape=jax.ShapeDtypeStruct((Q, NUM_CLS), jnp.float32),
    )(X, centroids, q2, c2.reshape(1, NUM_CLS * N_CLUSTERS))

    knn_pred = jnp.argmax(knn_scores, axis=1).astype(jnp.int32)
    return knn_pred, knn_scores, cent_dist


# speedup vs baseline: 11.6139x; 1.2207x over previous
import jax
import jax.numpy as jnp
from jax import lax
from jax.experimental import pallas as pl
from jax.experimental.pallas import tpu as pltpu
from jax.experimental.pallas import tpu_sc as plsc

K_NEIGH = 16
NUM_CLS = 4
N_CLUSTERS = 2
R_EXP = 1.0
EPS = 1e-8

Q = 4096
D = 128
M = 100000
B = 2048
NBLK = 49
MP = NBLK * B
QT = 256
GSZ = 128
NG = MP // GSZ
NCAND = K_NEIGH * GSZ
NROWS = K_NEIGH * Q
GW = 128

BIGF = 3e38
BIGI = 2**31 - 1


def _dist_colmin_kernel(x_ref, xm_ref, q2_ref, m2_ref, d_ref, cm_ref):
    dot = lax.dot_general(
        x_ref[...], xm_ref[...],
        dimension_numbers=(((1,), (1,)), ((), ())),
        preferred_element_type=jnp.float32,
    )
    d2 = (q2_ref[...] + m2_ref[...]) - 2.0 * dot
    for c in range(B // 128):
        d_ref[:, c, :, :] = d2[:, c * 128:(c + 1) * 128].reshape(QT // 8, 8, 128)
    cm = jnp.concatenate(
        [jnp.min(d2[:, c * 128:(c + 1) * 128], axis=1, keepdims=True)
         for c in range(B // 128)], axis=1)
    cm_ref[...] = cm.T


def _group_topk_kernel(cm_ref, g_ref):
    val = cm_ref[...]
    ridx = lax.broadcasted_iota(jnp.int32, (NG, QT), 0)
    for p in range(K_NEIGH):
        m = jnp.min(val, axis=0, keepdims=True)
        elig = val <= m
        ai = jnp.min(jnp.where(elig, ridx, BIGI), axis=0, keepdims=True)
        g_ref[p, :] = ai[0, :]
        val = jnp.where(ridx == ai, BIGF, val)


def _cand_topk_kernel(cv_ref, g_ref, nd_ref, js_ref):
    val = cv_ref[...]
    lane = lax.broadcasted_iota(jnp.int32, (QT, 128), 1)
    jv = jnp.concatenate(
        [g_ref[:, p:p + 1] * 128 + lane for p in range(K_NEIGH)],
        axis=1)
    for p in range(K_NEIGH):
        m = jnp.min(val, axis=1, keepdims=True)
        elig = val <= m
        aj = jnp.min(jnp.where(elig, jv, BIGI), axis=1, keepdims=True)
        nd_ref[:, p] = m[:, 0]
        js_ref[:, p] = aj[:, 0]
        val = jnp.where(jv == aj, BIGF, val)


def _cent_kernel(x_ref, c_ref, q2_ref, c2_ref, o_ref):
    dot = lax.dot_general(
        x_ref[...], c_ref[...],
        dimension_numbers=(((1,), (1,)), ((), ())),
        preferred_element_type=jnp.float32,
    )
    dc = (q2_ref[...] + c2_ref[...]) - 2.0 * dot
    dc = dc.reshape(x_ref.shape[0], NUM_CLS, N_CLUSTERS)
    o_ref[...] = jnp.min(dc, axis=2)


def _gather_rows(d2_rows, row_ids):
    mesh = plsc.VectorSubcoreMesh(core_axis_name="c", subcore_axis_name="s")

    @pl.kernel(out_type=jax.ShapeDtypeStruct((NROWS, GSZ), jnp.float32),
               mesh=mesh)
    def sc_gather(x_hbm, i_hbm, o_hbm):
        def body(i_vmem, o_vmem):
            pltpu.sync_copy(x_hbm.at[i_vmem.at[0]], o_vmem)

        pltpu.emit_pipeline(
            body,
            grid=(NROWS // GW,),
            in_specs=[pl.BlockSpec((1, GW), lambda i: (0, i))],
            out_specs=[pl.BlockSpec((GW, GSZ), lambda i: (i, 0))],
            core_axis_name=("c", "s"),
            dimension_semantics=(pltpu.PARALLEL,),
        )(i_hbm, o_hbm)

    return sc_gather(d2_rows, row_ids)


def kernel(X, X_mem, y_mem, centroids):
    q2 = jnp.sum(X * X, axis=1, keepdims=True)
    m2 = jnp.sum(X_mem * X_mem, axis=1)
    m2p = jnp.full((MP,), 1e30, dtype=jnp.float32).at[:M].set(m2)
    xmp = jnp.zeros((MP, D), dtype=jnp.float32).at[:M].set(X_mem)

    d2, colmin_t = pl.pallas_call(
        _dist_colmin_kernel,
        grid=(Q // QT, NBLK),
        in_specs=[
            pl.BlockSpec((QT, D), lambda i, j: (i, 0)),
            pl.BlockSpec((B, D), lambda i, j: (j, 0)),
            pl.BlockSpec((QT, 1), lambda i, j: (i, 0)),
            pl.BlockSpec((1, B), lambda i, j: (0, j)),
        ],
        out_specs=[
            pl.BlockSpec((QT // 8, B // 128, 8, 128), lambda i, j: (i, j, 0, 0)),
            pl.BlockSpec((B // 128, QT), lambda i, j: (j, i)),
        ],
        out_shape=[
            jax.ShapeDtypeStruct((Q // 8, NG, 8, 128), jnp.float32),
            jax.ShapeDtypeStruct((NG, Q), jnp.float32),
        ],
    )(X, xmp, q2, m2p.reshape(1, MP))

    gids_t = pl.pallas_call(
        _group_topk_kernel,
        grid=(Q // QT,),
        in_specs=[pl.BlockSpec((NG, QT), lambda i: (0, i))],
        out_specs=pl.BlockSpec((K_NEIGH, QT), lambda i: (0, i)),
        out_shape=jax.ShapeDtypeStruct((K_NEIGH, Q), jnp.int32),
    )(colmin_t)
    gids = gids_t.T

    qv = jnp.arange(Q, dtype=jnp.int32)[:, None]
    row_ids = (qv // 8) * (NG * 8) + gids * 8 + (qv % 8)
    cand_v = _gather_rows(
        d2.reshape(Q * MP // GSZ, GSZ), row_ids.reshape(1, NROWS)
    ).reshape(Q, NCAND)

    ndv, jsel = pl.pallas_call(
        _cand_topk_kernel,
        grid=(Q // QT,),
        in_specs=[
            pl.BlockSpec((QT, NCAND), lambda i: (i, 0)),
            pl.BlockSpec((QT, K_NEIGH), lambda i: (i, 0)),
        ],
        out_specs=[
            pl.BlockSpec((QT, K_NEIGH), lambda i: (i, 0)),
            pl.BlockSpec((QT, K_NEIGH), lambda i: (i, 0)),
        ],
        out_shape=[
            jax.ShapeDtypeStruct((Q, K_NEIGH), jnp.float32),
            jax.ShapeDtypeStruct((Q, K_NEIGH), jnp.int32),
        ],
    )(cand_v, gids)

    nd = jnp.maximum(ndv, 0.0)
    labels = jnp.take(y_mem, jsel, axis=0)
    w = 1.0 / (nd + EPS) ** R_EXP
    onehot = jax.nn.one_hot(labels, NUM_CLS, dtype=X.dtype)
    knn_scores = jnp.sum(w[..., None] * onehot, axis=1)
    knn_scores = knn_scores / (jnp.sum(w, axis=1, keepdims=True) + EPS)

    c2 = jnp.sum(centroids * centroids, axis=1)
    cent_dist = pl.pallas_call(
        _cent_kernel,
        grid=(Q // QT,),
        in_specs=[
            pl.BlockSpec((QT, D), lambda i: (i, 0)),
            pl.BlockSpec((NUM_CLS * N_CLUSTERS, D), lambda i: (0, 0)),
            pl.BlockSpec((QT, 1), lambda i: (i, 0)),
            pl.BlockSpec((1, NUM_CLS * N_CLUSTERS), lambda i: (0, 0)),
        ],
        out_specs=pl.BlockSpec((QT, NUM_CLS), lambda i: (i, 0)),
        out_shape=jax.ShapeDtypeStruct((Q, NUM_CLS), jnp.float32),
    )(X, centroids, q2, c2.reshape(1, NUM_CLS * N_CLUSTERS))

    knn_pred = jnp.argmax(knn_scores, axis=1).astype(jnp.int32)
    return knn_pred, knn_scores, cent_dist


# speedup vs baseline: 12.9723x; 1.1170x over previous
import jax
import jax.numpy as jnp
from jax import lax
from jax.experimental import pallas as pl
from jax.experimental.pallas import tpu as pltpu
from jax.experimental.pallas import tpu_sc as plsc

K_NEIGH = 16
NUM_CLS = 4
N_CLUSTERS = 2
R_EXP = 1.0
EPS = 1e-8

Q = 4096
D = 128
M = 100000
B = 2048
NBLK = 49
MP = NBLK * B
QT = 256
GSZ = 128
NG = MP // GSZ
NCAND = K_NEIGH * GSZ
NROWS = K_NEIGH * Q
GW = 128

BIGF = 3e38
BIGI = 2**31 - 1


def _dist_colmin_kernel(x_ref, xm_ref, q2_ref, m2_ref, d_ref, cm_ref):
    dot = lax.dot_general(
        x_ref[...], xm_ref[...],
        dimension_numbers=(((1,), (1,)), ((), ())),
        preferred_element_type=jnp.float32,
    )
    d2 = (q2_ref[...] + m2_ref[...]) - 2.0 * dot
    for c in range(B // 128):
        d_ref[:, c, :, :] = d2[:, c * 128:(c + 1) * 128].reshape(QT // 8, 8, 128)
    cm = jnp.concatenate(
        [jnp.min(d2[:, c * 128:(c + 1) * 128], axis=1, keepdims=True)
         for c in range(B // 128)], axis=1)
    cm_ref[...] = cm.T


def _group_topk_kernel(cm_ref, g_ref):
    val = cm_ref[...]
    ridx = lax.broadcasted_iota(jnp.int32, (NG, QT), 0)
    for p in range(K_NEIGH):
        m = jnp.min(val, axis=0, keepdims=True)
        elig = val <= m
        ai = jnp.min(jnp.where(elig, ridx, BIGI), axis=0, keepdims=True)
        g_ref[p, :] = ai[0, :]
        val = jnp.where(ridx == ai, BIGF, val)


def _cand_topk_kernel(cv_ref, g_ref, nd_ref, js_ref):
    val = cv_ref[...]
    lane = lax.broadcasted_iota(jnp.int32, (QT, 128), 1)
    jv = jnp.concatenate(
        [g_ref[:, p:p + 1] * 128 + lane for p in range(K_NEIGH)],
        axis=1)
    for p in range(K_NEIGH):
        m = jnp.min(val, axis=1, keepdims=True)
        elig = val <= m
        aj = jnp.min(jnp.where(elig, jv, BIGI), axis=1, keepdims=True)
        nd_ref[:, p] = m[:, 0]
        js_ref[:, p] = aj[:, 0]
        val = jnp.where(jv == aj, BIGF, val)


def _cent_kernel(x_ref, c_ref, q2_ref, c2_ref, o_ref):
    dot = lax.dot_general(
        x_ref[...], c_ref[...],
        dimension_numbers=(((1,), (1,)), ((), ())),
        preferred_element_type=jnp.float32,
    )
    dc = (q2_ref[...] + c2_ref[...]) - 2.0 * dot
    dc = dc.reshape(x_ref.shape[0], NUM_CLS, N_CLUSTERS)
    o_ref[...] = jnp.min(dc, axis=2)


def _gather_rows(d2_rows, row_ids):
    mesh = plsc.VectorSubcoreMesh(core_axis_name="c", subcore_axis_name="s")

    @pl.kernel(out_type=jax.ShapeDtypeStruct((NROWS, GSZ), jnp.float32),
               mesh=mesh)
    def sc_gather(x_hbm, i_hbm, o_hbm):
        def body(i_vmem, o_vmem):
            pltpu.sync_copy(x_hbm.at[i_vmem.at[0]], o_vmem)

        pltpu.emit_pipeline(
            body,
            grid=(NROWS // GW,),
            in_specs=[pl.BlockSpec((1, GW), lambda i: (0, i))],
            out_specs=[pl.BlockSpec((GW, GSZ), lambda i: (i, 0))],
            core_axis_name=("c", "s"),
            dimension_semantics=(pltpu.PARALLEL,),
        )(i_hbm, o_hbm)

    return sc_gather(d2_rows, row_ids)


def kernel(X, X_mem, y_mem, centroids):
    q2 = jnp.sum(X * X, axis=1, keepdims=True)
    m2 = jnp.sum(X_mem * X_mem, axis=1)
    m2p = jnp.full((MP,), 1e30, dtype=jnp.float32).at[:M].set(m2)
    xmp = jnp.zeros((MP, D), dtype=jnp.float32).at[:M].set(X_mem)

    d2, colmin_t = pl.pallas_call(
        _dist_colmin_kernel,
        grid=(NBLK, Q // QT),
        in_specs=[
            pl.BlockSpec((QT, D), lambda j, i: (i, 0)),
            pl.BlockSpec((B, D), lambda j, i: (j, 0)),
            pl.BlockSpec((QT, 1), lambda j, i: (i, 0)),
            pl.BlockSpec((1, B), lambda j, i: (0, j)),
        ],
        out_specs=[
            pl.BlockSpec((QT // 8, B // 128, 8, 128), lambda j, i: (i, j, 0, 0)),
            pl.BlockSpec((B // 128, QT), lambda j, i: (j, i)),
        ],
        out_shape=[
            jax.ShapeDtypeStruct((Q // 8, NG, 8, 128), jnp.float32),
            jax.ShapeDtypeStruct((NG, Q), jnp.float32),
        ],
    )(X, xmp, q2, m2p.reshape(1, MP))

    gids_t = pl.pallas_call(
        _group_topk_kernel,
        grid=(Q // QT,),
        in_specs=[pl.BlockSpec((NG, QT), lambda i: (0, i))],
        out_specs=pl.BlockSpec((K_NEIGH, QT), lambda i: (0, i)),
        out_shape=jax.ShapeDtypeStruct((K_NEIGH, Q), jnp.int32),
    )(colmin_t)
    gids = gids_t.T

    qv = jnp.arange(Q, dtype=jnp.int32)[:, None]
    row_ids = (qv // 8) * (NG * 8) + gids * 8 + (qv % 8)
    cand_v = _gather_rows(
        d2.reshape(Q * MP // GSZ, GSZ), row_ids.reshape(1, NROWS)
    ).reshape(Q, NCAND)

    ndv, jsel = pl.pallas_call(
        _cand_topk_kernel,
        grid=(Q // QT,),
        in_specs=[
            pl.BlockSpec((QT, NCAND), lambda i: (i, 0)),
            pl.BlockSpec((QT, K_NEIGH), lambda i: (i, 0)),
        ],
        out_specs=[
            pl.BlockSpec((QT, K_NEIGH), lambda i: (i, 0)),
            pl.BlockSpec((QT, K_NEIGH), lambda i: (i, 0)),
        ],
        out_shape=[
            jax.ShapeDtypeStruct((Q, K_NEIGH), jnp.float32),
            jax.ShapeDtypeStruct((Q, K_NEIGH), jnp.int32),
        ],
    )(cand_v, gids)

    nd = jnp.maximum(ndv, 0.0)
    labels = jnp.take(y_mem, jsel, axis=0)
    w = 1.0 / (nd + EPS) ** R_EXP
    onehot = jax.nn.one_hot(labels, NUM_CLS, dtype=X.dtype)
    knn_scores = jnp.sum(w[..., None] * onehot, axis=1)
    knn_scores = knn_scores / (jnp.sum(w, axis=1, keepdims=True) + EPS)

    c2 = jnp.sum(centroids * centroids, axis=1)
    cent_dist = pl.pallas_call(
        _cent_kernel,
        grid=(Q // QT,),
        in_specs=[
            pl.BlockSpec((QT, D), lambda i: (i, 0)),
            pl.BlockSpec((NUM_CLS * N_CLUSTERS, D), lambda i: (0, 0)),
            pl.BlockSpec((QT, 1), lambda i: (i, 0)),
            pl.BlockSpec((1, NUM_CLS * N_CLUSTERS), lambda i: (0, 0)),
        ],
        out_specs=pl.BlockSpec((QT, NUM_CLS), lambda i: (i, 0)),
        out_shape=jax.ShapeDtypeStruct((Q, NUM_CLS), jnp.float32),
    )(X, centroids, q2, c2.reshape(1, NUM_CLS * N_CLUSTERS))

    knn_pred = jnp.argmax(knn_scores, axis=1).astype(jnp.int32)
    return knn_pred, knn_scores, cent_dist


# speedup vs baseline: 17.5395x; 1.3521x over previous
import jax
import jax.numpy as jnp
from jax import lax
from jax.experimental import pallas as pl
from jax.experimental.pallas import tpu as pltpu
from jax.experimental.pallas import tpu_sc as plsc

K_NEIGH = 16
NUM_CLS = 4
N_CLUSTERS = 2
R_EXP = 1.0
EPS = 1e-8

Q = 4096
D = 128
M = 100000
B = 4096
NBLK = 25
MP = NBLK * B
QT = 512
GSZ = 128
NG = MP // GSZ
NCAND = K_NEIGH * GSZ
NROWS = K_NEIGH * Q
GW = 128

BIGF = 3e38
BIGI = 2**31 - 1


def _dist_colmin_kernel(x_ref, xm_ref, q2_ref, m2_ref, d_ref, cm_ref):
    dot = lax.dot_general(
        x_ref[...], xm_ref[...],
        dimension_numbers=(((1,), (1,)), ((), ())),
        preferred_element_type=jnp.float32,
    )
    d2 = (q2_ref[...] + m2_ref[...]) - 2.0 * dot
    for c in range(B // 128):
        d_ref[:, c, :, :] = d2[:, c * 128:(c + 1) * 128].reshape(QT // 8, 8, 128)
    cm = jnp.concatenate(
        [jnp.min(d2[:, c * 128:(c + 1) * 128], axis=1, keepdims=True)
         for c in range(B // 128)], axis=1)
    cm_ref[...] = cm.T


def _group_topk_kernel(cm_ref, g_ref):
    val = cm_ref[...]
    ridx = lax.broadcasted_iota(jnp.int32, (NG, QT), 0)
    for p in range(K_NEIGH):
        m = jnp.min(val, axis=0, keepdims=True)
        elig = val <= m
        ai = jnp.min(jnp.where(elig, ridx, BIGI), axis=0, keepdims=True)
        g_ref[p, :] = ai[0, :]
        val = jnp.where(ridx == ai, BIGF, val)


def _cand_topk_kernel(cv_ref, g_ref, nd_ref, js_ref):
    val = cv_ref[...]
    lane = lax.broadcasted_iota(jnp.int32, (QT, 128), 1)
    jv = jnp.concatenate(
        [g_ref[:, p:p + 1] * 128 + lane for p in range(K_NEIGH)],
        axis=1)
    for p in range(K_NEIGH):
        m = jnp.min(val, axis=1, keepdims=True)
        elig = val <= m
        aj = jnp.min(jnp.where(elig, jv, BIGI), axis=1, keepdims=True)
        nd_ref[:, p] = m[:, 0]
        js_ref[:, p] = aj[:, 0]
        val = jnp.where(jv == aj, BIGF, val)


def _cent_kernel(x_ref, c_ref, q2_ref, c2_ref, o_ref):
    dot = lax.dot_general(
        x_ref[...], c_ref[...],
        dimension_numbers=(((1,), (1,)), ((), ())),
        preferred_element_type=jnp.float32,
    )
    dc = (q2_ref[...] + c2_ref[...]) - 2.0 * dot
    dc = dc.reshape(x_ref.shape[0], NUM_CLS, N_CLUSTERS)
    o_ref[...] = jnp.min(dc, axis=2)


def _gather_rows(d2_rows, row_ids):
    mesh = plsc.VectorSubcoreMesh(core_axis_name="c", subcore_axis_name="s")

    @pl.kernel(out_type=jax.ShapeDtypeStruct((NROWS, GSZ), jnp.float32),
               mesh=mesh)
    def sc_gather(x_hbm, i_hbm, o_hbm):
        def body(i_vmem, o_vmem):
            pltpu.sync_copy(x_hbm.at[i_vmem.at[0]], o_vmem)

        pltpu.emit_pipeline(
            body,
            grid=(NROWS // GW,),
            in_specs=[pl.BlockSpec((1, GW), lambda i: (0, i))],
            out_specs=[pl.BlockSpec((GW, GSZ), lambda i: (i, 0))],
            core_axis_name=("c", "s"),
            dimension_semantics=(pltpu.PARALLEL,),
        )(i_hbm, o_hbm)

    return sc_gather(d2_rows, row_ids)


def kernel(X, X_mem, y_mem, centroids):
    q2 = jnp.sum(X * X, axis=1, keepdims=True)
    m2 = jnp.sum(X_mem * X_mem, axis=1)
    m2p = jnp.full((MP,), 1e30, dtype=jnp.float32).at[:M].set(m2)
    xmp = jnp.zeros((MP, D), dtype=jnp.float32).at[:M].set(X_mem)

    d2, colmin_t = pl.pallas_call(
        _dist_colmin_kernel,
        grid=(NBLK, Q // QT),
        in_specs=[
            pl.BlockSpec((QT, D), lambda j, i: (i, 0)),
            pl.BlockSpec((B, D), lambda j, i: (j, 0)),
            pl.BlockSpec((QT, 1), lambda j, i: (i, 0)),
            pl.BlockSpec((1, B), lambda j, i: (0, j)),
        ],
        out_specs=[
            pl.BlockSpec((QT // 8, B // 128, 8, 128), lambda j, i: (i, j, 0, 0)),
            pl.BlockSpec((B // 128, QT), lambda j, i: (j, i)),
        ],
        out_shape=[
            jax.ShapeDtypeStruct((Q // 8, NG, 8, 128), jnp.float32),
            jax.ShapeDtypeStruct((NG, Q), jnp.float32),
        ],
    )(X, xmp, q2, m2p.reshape(1, MP))

    gids_t = pl.pallas_call(
        _group_topk_kernel,
        grid=(Q // QT,),
        in_specs=[pl.BlockSpec((NG, QT), lambda i: (0, i))],
        out_specs=pl.BlockSpec((K_NEIGH, QT), lambda i: (0, i)),
        out_shape=jax.ShapeDtypeStruct((K_NEIGH, Q), jnp.int32),
    )(colmin_t)
    gids = gids_t.T

    qv = jnp.arange(Q, dtype=jnp.int32)[:, None]
    row_ids = (qv // 8) * (NG * 8) + gids * 8 + (qv % 8)
    cand_v = _gather_rows(
        d2.reshape(Q * MP // GSZ, GSZ), row_ids.reshape(1, NROWS)
    ).reshape(Q, NCAND)

    ndv, jsel = pl.pallas_call(
        _cand_topk_kernel,
        grid=(Q // QT,),
        in_specs=[
            pl.BlockSpec((QT, NCAND), lambda i: (i, 0)),
            pl.BlockSpec((QT, K_NEIGH), lambda i: (i, 0)),
        ],
        out_specs=[
            pl.BlockSpec((QT, K_NEIGH), lambda i: (i, 0)),
            pl.BlockSpec((QT, K_NEIGH), lambda i: (i, 0)),
        ],
        out_shape=[
            jax.ShapeDtypeStruct((Q, K_NEIGH), jnp.float32),
            jax.ShapeDtypeStruct((Q, K_NEIGH), jnp.int32),
        ],
    )(cand_v, gids)

    nd = jnp.maximum(ndv, 0.0)
    labels = jnp.take(y_mem, jsel, axis=0)
    w = 1.0 / (nd + EPS) ** R_EXP
    onehot = jax.nn.one_hot(labels, NUM_CLS, dtype=X.dtype)
    knn_scores = jnp.sum(w[..., None] * onehot, axis=1)
    knn_scores = knn_scores / (jnp.sum(w, axis=1, keepdims=True) + EPS)

    c2 = jnp.sum(centroids * centroids, axis=1)
    cent_dist = pl.pallas_call(
        _cent_kernel,
        grid=(Q // QT,),
        in_specs=[
            pl.BlockSpec((QT, D), lambda i: (i, 0)),
            pl.BlockSpec((NUM_CLS * N_CLUSTERS, D), lambda i: (0, 0)),
            pl.BlockSpec((QT, 1), lambda i: (i, 0)),
            pl.BlockSpec((1, NUM_CLS * N_CLUSTERS), lambda i: (0, 0)),
        ],
        out_specs=pl.BlockSpec((QT, NUM_CLS), lambda i: (i, 0)),
        out_shape=jax.ShapeDtypeStruct((Q, NUM_CLS), jnp.float32),
    )(X, centroids, q2, c2.reshape(1, NUM_CLS * N_CLUSTERS))

    knn_pred = jnp.argmax(knn_scores, axis=1).astype(jnp.int32)
    return knn_pred, knn_scores, cent_dist


# speedup vs baseline: 18.1203x; 1.0331x over previous
import jax
import jax.numpy as jnp
from jax import lax
from jax.experimental import pallas as pl
from jax.experimental.pallas import tpu as pltpu
from jax.experimental.pallas import tpu_sc as plsc

K_NEIGH = 16
NUM_CLS = 4
N_CLUSTERS = 2
R_EXP = 1.0
EPS = 1e-8

Q = 4096
D = 128
M = 100000
B = 8192
NBLK = 13
MP = NBLK * B
QT = 512
GSZ = 128
NG = MP // GSZ
NCAND = K_NEIGH * GSZ
NROWS = K_NEIGH * Q
GW = 128

BIGF = 3e38
BIGI = 2**31 - 1


def _dist_colmin_kernel(x_ref, xm_ref, q2_ref, m2_ref, d_ref, cm_ref):
    dot = lax.dot_general(
        x_ref[...], xm_ref[...],
        dimension_numbers=(((1,), (1,)), ((), ())),
        preferred_element_type=jnp.float32,
    )
    d2 = (q2_ref[...] + m2_ref[...]) - 2.0 * dot
    for c in range(B // 128):
        d_ref[:, c, :, :] = d2[:, c * 128:(c + 1) * 128].reshape(QT // 8, 8, 128)
    cm = jnp.concatenate(
        [jnp.min(d2[:, c * 128:(c + 1) * 128], axis=1, keepdims=True)
         for c in range(B // 128)], axis=1)
    cm_ref[...] = cm.T


def _group_topk_kernel(cm_ref, g_ref):
    val = cm_ref[...]
    ridx = lax.broadcasted_iota(jnp.int32, (NG, QT), 0)
    for p in range(K_NEIGH):
        m = jnp.min(val, axis=0, keepdims=True)
        elig = val <= m
        ai = jnp.min(jnp.where(elig, ridx, BIGI), axis=0, keepdims=True)
        g_ref[p, :] = ai[0, :]
        val = jnp.where(ridx == ai, BIGF, val)


def _cand_topk_kernel(cv_ref, g_ref, nd_ref, js_ref):
    val = cv_ref[...]
    lane = lax.broadcasted_iota(jnp.int32, (QT, 128), 1)
    jv = jnp.concatenate(
        [g_ref[:, p:p + 1] * 128 + lane for p in range(K_NEIGH)],
        axis=1)
    for p in range(K_NEIGH):
        m = jnp.min(val, axis=1, keepdims=True)
        elig = val <= m
        aj = jnp.min(jnp.where(elig, jv, BIGI), axis=1, keepdims=True)
        nd_ref[:, p] = m[:, 0]
        js_ref[:, p] = aj[:, 0]
        val = jnp.where(jv == aj, BIGF, val)


def _cent_kernel(x_ref, c_ref, q2_ref, c2_ref, o_ref):
    dot = lax.dot_general(
        x_ref[...], c_ref[...],
        dimension_numbers=(((1,), (1,)), ((), ())),
        preferred_element_type=jnp.float32,
    )
    dc = (q2_ref[...] + c2_ref[...]) - 2.0 * dot
    dc = dc.reshape(x_ref.shape[0], NUM_CLS, N_CLUSTERS)
    o_ref[...] = jnp.min(dc, axis=2)


def _gather_rows(d2_rows, row_ids):
    mesh = plsc.VectorSubcoreMesh(core_axis_name="c", subcore_axis_name="s")

    @pl.kernel(out_type=jax.ShapeDtypeStruct((NROWS, GSZ), jnp.float32),
               mesh=mesh)
    def sc_gather(x_hbm, i_hbm, o_hbm):
        def body(i_vmem, o_vmem):
            pltpu.sync_copy(x_hbm.at[i_vmem.at[0]], o_vmem)

        pltpu.emit_pipeline(
            body,
            grid=(NROWS // GW,),
            in_specs=[pl.BlockSpec((1, GW), lambda i: (0, i))],
            out_specs=[pl.BlockSpec((GW, GSZ), lambda i: (i, 0))],
            core_axis_name=("c", "s"),
            dimension_semantics=(pltpu.PARALLEL,),
        )(i_hbm, o_hbm)

    return sc_gather(d2_rows, row_ids)


def kernel(X, X_mem, y_mem, centroids):
    q2 = jnp.sum(X * X, axis=1, keepdims=True)
    m2 = jnp.sum(X_mem * X_mem, axis=1)
    m2p = jnp.full((MP,), 1e30, dtype=jnp.float32).at[:M].set(m2)
    xmp = jnp.zeros((MP, D), dtype=jnp.float32).at[:M].set(X_mem)

    d2, colmin_t = pl.pallas_call(
        _dist_colmin_kernel,
        grid=(NBLK, Q // QT),
        in_specs=[
            pl.BlockSpec((QT, D), lambda j, i: (i, 0)),
            pl.BlockSpec((B, D), lambda j, i: (j, 0)),
            pl.BlockSpec((QT, 1), lambda j, i: (i, 0)),
            pl.BlockSpec((1, B), lambda j, i: (0, j)),
        ],
        out_specs=[
            pl.BlockSpec((QT // 8, B // 128, 8, 128), lambda j, i: (i, j, 0, 0)),
            pl.BlockSpec((B // 128, QT), lambda j, i: (j, i)),
        ],
        out_shape=[
            jax.ShapeDtypeStruct((Q // 8, NG, 8, 128), jnp.float32),
            jax.ShapeDtypeStruct((NG, Q), jnp.float32),
        ],
    )(X, xmp, q2, m2p.reshape(1, MP))

    gids_t = pl.pallas_call(
        _group_topk_kernel,
        grid=(Q // QT,),
        in_specs=[pl.BlockSpec((NG, QT), lambda i: (0, i))],
        out_specs=pl.BlockSpec((K_NEIGH, QT), lambda i: (0, i)),
        out_shape=jax.ShapeDtypeStruct((K_NEIGH, Q), jnp.int32),
    )(colmin_t)
    gids = gids_t.T

    qv = jnp.arange(Q, dtype=jnp.int32)[:, None]
    row_ids = (qv // 8) * (NG * 8) + gids * 8 + (qv % 8)
    cand_v = _gather_rows(
        d2.reshape(Q * MP // GSZ, GSZ), row_ids.reshape(1, NROWS)
    ).reshape(Q, NCAND)

    ndv, jsel = pl.pallas_call(
        _cand_topk_kernel,
        grid=(Q // QT,),
        in_specs=[
            pl.BlockSpec((QT, NCAND), lambda i: (i, 0)),
            pl.BlockSpec((QT, K_NEIGH), lambda i: (i, 0)),
        ],
        out_specs=[
            pl.BlockSpec((QT, K_NEIGH), lambda i: (i, 0)),
            pl.BlockSpec((QT, K_NEIGH), lambda i: (i, 0)),
        ],
        out_shape=[
            jax.ShapeDtypeStruct((Q, K_NEIGH), jnp.float32),
            jax.ShapeDtypeStruct((Q, K_NEIGH), jnp.int32),
        ],
    )(cand_v, gids)

    nd = jnp.maximum(ndv, 0.0)
    labels = jnp.take(y_mem, jsel, axis=0)
    w = 1.0 / (nd + EPS) ** R_EXP
    onehot = jax.nn.one_hot(labels, NUM_CLS, dtype=X.dtype)
    knn_scores = jnp.sum(w[..., None] * onehot, axis=1)
    knn_scores = knn_scores / (jnp.sum(w, axis=1, keepdims=True) + EPS)

    c2 = jnp.sum(centroids * centroids, axis=1)
    cent_dist = pl.pallas_call(
        _cent_kernel,
        grid=(Q // QT,),
        in_specs=[
            pl.BlockSpec((QT, D), lambda i: (i, 0)),
            pl.BlockSpec((NUM_CLS * N_CLUSTERS, D), lambda i: (0, 0)),
            pl.BlockSpec((QT, 1), lambda i: (i, 0)),
            pl.BlockSpec((1, NUM_CLS * N_CLUSTERS), lambda i: (0, 0)),
        ],
        out_specs=pl.BlockSpec((QT, NUM_CLS), lambda i: (i, 0)),
        out_shape=jax.ShapeDtypeStruct((Q, NUM_CLS), jnp.float32),
    )(X, centroids, q2, c2.reshape(1, NUM_CLS * N_CLUSTERS))

    knn_pred = jnp.argmax(knn_scores, axis=1).astype(jnp.int32)
    return knn_pred, knn_scores, cent_dist
